# Initial kernel scaffold; baseline (speedup 1.0000x reference)
import jax
import jax.numpy as jnp
from jax.experimental import pallas as pl


def kernel(features, cluster):
    raise NotImplementedError("write your pallas kernel here")


# speedup vs baseline: 1.7164x; 1.7164x over previous
import jax
import jax.numpy as jnp
from jax import lax
from jax.experimental import pallas as pl
from jax.experimental.pallas import tpu as pltpu
from jax.experimental.pallas import tpu_sc as plsc

N_FINE = 50000
N_COARSE = 10000
D_FEAT = 512

NC, NS = 2, 16
NW = NC * NS

CW = 112
NCH_H = 14
CH_H = CW * NCH_H
HPAD = NW * CH_H
NBINS = 10240

GCH = 80
NCHUNK = N_FINE // GCH
KMAX = -(-NCHUNK // NW)


def _mesh():
    return plsc.VectorSubcoreMesh(core_axis_name="c", subcore_axis_name="s",
                                  num_cores=NC, num_subcores=NS)


def _hist_body(idx_hbm, out_hbm, idx_v, ones_v, zero_v, hist_sh):
    cid = lax.axis_index("c")
    sid = lax.axis_index("s")
    wid = cid * NS + sid

    def fill_ones(i, _):
        ones_v[0, pl.ds(i * 16, 16)] = jnp.ones((16,), jnp.float32)
        return 0

    lax.fori_loop(0, CW // 16, fill_ones, 0)

    @pl.when(sid == 0)
    def _():
        def fill_zero(i, _):
            zero_v[pl.ds(i * 16, 16)] = jnp.zeros((16,), jnp.float32)
            return 0

        lax.fori_loop(0, NBINS // 16, fill_zero, 0)
        pltpu.sync_copy(zero_v, hist_sh)

    base = wid * CH_H
    for j in range(NCH_H):
        pltpu.sync_copy(idx_hbm.at[pl.ds(base + j * CW, CW)], idx_v.at[j])
    plsc.subcore_barrier()
    for j in range(NCH_H):
        pltpu.sync_copy(ones_v.at[0], hist_sh.at[idx_v.at[j]], add=True)
    plsc.subcore_barrier()

    @pl.when(sid == 0)
    def _():
        pltpu.sync_copy(hist_sh, out_hbm.at[cid])


def _hist(idx_pad):
    k = pl.kernel(
        _hist_body,
        out_type=jax.ShapeDtypeStruct((NC, NBINS), jnp.float32),
        mesh=_mesh(),
        scratch_types=[
            pltpu.VMEM((NCH_H, CW), jnp.int32),
            pltpu.VMEM((1, CW), jnp.float32),
            pltpu.VMEM((NBINS,), jnp.float32),
            pltpu.VMEM_SHARED((NBINS,), jnp.float32),
        ],
    )
    return k(idx_pad)


BR = 2000


def _scale_body(f_ref, a_ref, b_ref, o_ref):
    o_ref[...] = f_ref[...] / jnp.maximum(a_ref[...] + b_ref[...], 1.0)


def _scale(features, pc0, pc1):
    return pl.pallas_call(
        _scale_body,
        out_shape=jax.ShapeDtypeStruct((N_COARSE, D_FEAT), jnp.float32),
        grid=(N_COARSE // BR,),
        in_specs=[
            pl.BlockSpec((BR, D_FEAT), lambda i: (i, 0)),
            pl.BlockSpec((BR, 1), lambda i: (i, 0)),
            pl.BlockSpec((BR, 1), lambda i: (i, 0)),
        ],
        out_specs=pl.BlockSpec((BR, D_FEAT), lambda i: (i, 0)),
    )(features, pc0, pc1)


def _gather_body(tab_hbm, idx_hbm, out_hbm, idx_v, rows_v, sem):
    cid = lax.axis_index("c")
    sid = lax.axis_index("s")
    wid = cid * NS + sid

    def step(k, _):
        c = k * NW + wid

        @pl.when(c < NCHUNK)
        def _():
            base = c * GCH
            pltpu.sync_copy(idx_hbm.at[pl.ds(base, GCH)], idx_v)
            pltpu.async_copy(tab_hbm.at[idx_v], rows_v, sem).wait()
            pltpu.sync_copy(rows_v, out_hbm.at[pl.ds(base, GCH)])

        return 0

    lax.fori_loop(0, KMAX, step, 0)


def _gather(scaled, idx):
    k = pl.kernel(
        _gather_body,
        out_type=jax.ShapeDtypeStruct((N_FINE, D_FEAT), jnp.float32),
        mesh=_mesh(),
        scratch_types=[
            pltpu.VMEM((GCH,), jnp.int32),
            pltpu.VMEM((GCH, D_FEAT), jnp.float32),
            pltpu.SemaphoreType.DMA,
        ],
    )
    return k(scaled, idx)


def kernel(features, cluster):
    idx = cluster.astype(jnp.int32)
    idx_pad = jnp.concatenate(
        [idx, jnp.full((HPAD - N_FINE,), N_COARSE, jnp.int32)])
    partials = _hist(idx_pad)
    pc0 = partials[0, :N_COARSE].reshape(N_COARSE, 1)
    pc1 = partials[1, :N_COARSE].reshape(N_COARSE, 1)
    scaled = _scale(features, pc0, pc1)
    return _gather(scaled, idx)


# speedup vs baseline: 1.9800x; 1.1536x over previous
import jax
import jax.numpy as jnp
from jax import lax
from jax.experimental import pallas as pl
from jax.experimental.pallas import tpu as pltpu
from jax.experimental.pallas import tpu_sc as plsc

N_FINE = 50000
N_COARSE = 10000
D_FEAT = 512

NC, NS = 2, 16
NW = NC * NS

CW = 112
NCH_H = 14
CH_H = CW * NCH_H
HPAD = NW * CH_H
NBINS = 10240

GCH = 80
NCHUNK = N_FINE // GCH
KMAX = -(-NCHUNK // NW)


def _mesh():
    return plsc.VectorSubcoreMesh(core_axis_name="c", subcore_axis_name="s",
                                  num_cores=NC, num_subcores=NS)


def _hist_body(idx_hbm, out_hbm, idx_v, ones_v, zero_v, hist_sh):
    cid = lax.axis_index("c")
    sid = lax.axis_index("s")
    wid = cid * NS + sid

    def fill_ones(i, _):
        ones_v[0, pl.ds(i * 16, 16)] = jnp.ones((16,), jnp.float32)
        return 0

    lax.fori_loop(0, CW // 16, fill_ones, 0)

    @pl.when(sid == 0)
    def _():
        def fill_zero(i, _):
            zero_v[pl.ds(i * 16, 16)] = jnp.zeros((16,), jnp.float32)
            return 0

        lax.fori_loop(0, NBINS // 16, fill_zero, 0)
        pltpu.sync_copy(zero_v, hist_sh)

    base = wid * CH_H
    for j in range(NCH_H):
        pltpu.sync_copy(idx_hbm.at[pl.ds(base + j * CW, CW)], idx_v.at[j])
    plsc.subcore_barrier()
    for j in range(NCH_H):
        pltpu.sync_copy(ones_v.at[0], hist_sh.at[idx_v.at[j]], add=True)
    plsc.subcore_barrier()

    @pl.when(sid == 0)
    def _():
        pltpu.sync_copy(hist_sh, out_hbm.at[cid])


def _hist(idx_pad):
    k = pl.kernel(
        _hist_body,
        out_type=jax.ShapeDtypeStruct((NC, NBINS), jnp.float32),
        mesh=_mesh(),
        scratch_types=[
            pltpu.VMEM((NCH_H, CW), jnp.int32),
            pltpu.VMEM((1, CW), jnp.float32),
            pltpu.VMEM((NBINS,), jnp.float32),
            pltpu.VMEM_SHARED((NBINS,), jnp.float32),
        ],
    )
    return k(idx_pad)


BR = 2000


def _scale_body(f_ref, a_ref, b_ref, o_ref):
    o_ref[...] = f_ref[...] / jnp.maximum(a_ref[...] + b_ref[...], 1.0)


def _scale(features, pc0, pc1):
    return pl.pallas_call(
        _scale_body,
        out_shape=jax.ShapeDtypeStruct((N_COARSE, D_FEAT), jnp.float32),
        grid=(N_COARSE // BR,),
        in_specs=[
            pl.BlockSpec((BR, D_FEAT), lambda i: (i, 0)),
            pl.BlockSpec((BR, 1), lambda i: (i, 0)),
            pl.BlockSpec((BR, 1), lambda i: (i, 0)),
        ],
        out_specs=pl.BlockSpec((BR, D_FEAT), lambda i: (i, 0)),
    )(features, pc0, pc1)


def _gather_body(tab_hbm, idx_hbm, out_hbm, idx_v, rows_v, sems):
    cid = lax.axis_index("c")
    sid = lax.axis_index("s")
    wid = cid * NS + sid

    def start(k, b):
        c = k * NW + wid

        @pl.when(c < NCHUNK)
        def _():
            pltpu.sync_copy(idx_hbm.at[pl.ds(c * GCH, GCH)], idx_v.at[b])
            pltpu.async_copy(tab_hbm.at[idx_v.at[b]], rows_v.at[b], sems.at[b])

    start(0, 0)
    start(1, 1)

    @pl.loop(0, KMAX, step=2)
    def _(k):
        for b in range(2):
            kk = k + b
            c = kk * NW + wid

            @pl.when(c < NCHUNK)
            def _():
                pltpu.make_async_copy(
                    tab_hbm.at[idx_v.at[b]], rows_v.at[b], sems.at[b]).wait()
                pltpu.sync_copy(rows_v.at[b], out_hbm.at[pl.ds(c * GCH, GCH)])

            start(kk + 2, b)


def _gather(scaled, idx):
    k = pl.kernel(
        _gather_body,
        out_type=jax.ShapeDtypeStruct((N_FINE, D_FEAT), jnp.float32),
        mesh=_mesh(),
        scratch_types=[
            pltpu.VMEM((2, GCH), jnp.int32),
            pltpu.VMEM((2, GCH, D_FEAT), jnp.float32),
            pltpu.SemaphoreType.DMA((2,)),
        ],
    )
    return k(scaled, idx)


def kernel(features, cluster):
    idx = cluster.astype(jnp.int32)
    idx_pad = jnp.concatenate(
        [idx, jnp.full((HPAD - N_FINE,), N_COARSE, jnp.int32)])
    partials = _hist(idx_pad)
    pc0 = partials[0, :N_COARSE].reshape(N_COARSE, 1)
    pc1 = partials[1, :N_COARSE].reshape(N_COARSE, 1)
    scaled = _scale(features, pc0, pc1)
    return _gather(scaled, idx)


# speedup vs baseline: 1.9886x; 1.0043x over previous
import jax
import jax.numpy as jnp
from jax import lax
from jax.experimental import pallas as pl
from jax.experimental.pallas import tpu as pltpu
from jax.experimental.pallas import tpu_sc as plsc

N_FINE = 50000
N_COARSE = 10000
D_FEAT = 512

NC, NS = 2, 16
NW = NC * NS

CW = 112
NCH_H = 14
CH_H = CW * NCH_H
HPAD = NW * CH_H
NBINS = 10240

GCH = 80
NCHUNK = N_FINE // GCH
KMAX = -(-NCHUNK // NW)


def _mesh():
    return plsc.VectorSubcoreMesh(core_axis_name="c", subcore_axis_name="s",
                                  num_cores=NC, num_subcores=NS)


def _hist_body(idx_hbm, out_hbm, idx_v, ones_v, zero_v, hist_sh):
    cid = lax.axis_index("c")
    sid = lax.axis_index("s")
    wid = cid * NS + sid

    def fill_ones(i, _):
        ones_v[0, pl.ds(i * 16, 16)] = jnp.ones((16,), jnp.float32)
        return 0

    lax.fori_loop(0, CW // 16, fill_ones, 0)

    @pl.when(sid == 0)
    def _():
        def fill_zero(i, _):
            zero_v[pl.ds(i * 16, 16)] = jnp.zeros((16,), jnp.float32)
            return 0

        lax.fori_loop(0, NBINS // 16, fill_zero, 0)
        pltpu.sync_copy(zero_v, hist_sh)

    base = wid * CH_H
    for j in range(NCH_H):
        pltpu.sync_copy(idx_hbm.at[pl.ds(base + j * CW, CW)], idx_v.at[j])
    plsc.subcore_barrier()
    for j in range(NCH_H):
        pltpu.sync_copy(ones_v.at[0], hist_sh.at[idx_v.at[j]], add=True)
    plsc.subcore_barrier()

    @pl.when(sid == 0)
    def _():
        pltpu.sync_copy(hist_sh, out_hbm.at[cid])


def _hist(idx_pad):
    k = pl.kernel(
        _hist_body,
        out_type=jax.ShapeDtypeStruct((NC, NBINS), jnp.float32),
        mesh=_mesh(),
        scratch_types=[
            pltpu.VMEM((NCH_H, CW), jnp.int32),
            pltpu.VMEM((1, CW), jnp.float32),
            pltpu.VMEM((NBINS,), jnp.float32),
            pltpu.VMEM_SHARED((NBINS,), jnp.float32),
        ],
    )
    return k(idx_pad)


BR = 2000


def _scale_body(f_ref, a_ref, b_ref, o_ref):
    o_ref[...] = f_ref[...] / jnp.maximum(a_ref[...] + b_ref[...], 1.0)


def _scale(features, pc0, pc1):
    return pl.pallas_call(
        _scale_body,
        out_shape=jax.ShapeDtypeStruct((N_COARSE, D_FEAT), jnp.float32),
        grid=(N_COARSE // BR,),
        in_specs=[
            pl.BlockSpec((BR, D_FEAT), lambda i: (i, 0)),
            pl.BlockSpec((BR, 1), lambda i: (i, 0)),
            pl.BlockSpec((BR, 1), lambda i: (i, 0)),
        ],
        out_specs=pl.BlockSpec((BR, D_FEAT), lambda i: (i, 0)),
    )(features, pc0, pc1)


NBUF = 3


def _gather_body(tab_hbm, idx_hbm, out_hbm, idx_all, rows_v, gsems, osems):
    cid = lax.axis_index("c")
    sid = lax.axis_index("s")
    wid = cid * NS + sid
    start_c = wid * 19 + jnp.minimum(wid, 17)
    k_w = jnp.where(wid < 17, KMAX, KMAX - 1)

    pltpu.sync_copy(idx_hbm.at[pl.ds(start_c * GCH, KMAX * GCH)], idx_all)

    def start_gather(k, b):
        pltpu.async_copy(
            tab_hbm.at[idx_all.at[pl.ds(k * GCH, GCH)]], rows_v.at[b],
            gsems.at[b])

    def wait_gather(k, b):
        pltpu.make_async_copy(
            tab_hbm.at[idx_all.at[pl.ds(k * GCH, GCH)]], rows_v.at[b],
            gsems.at[b]).wait()

    def start_scatter(k, b):
        pltpu.async_copy(
            rows_v.at[b], out_hbm.at[pl.ds((start_c + k) * GCH, GCH)],
            osems.at[b])

    def drain_scatter(b):
        pltpu.make_async_copy(
            rows_v.at[b], out_hbm.at[pl.ds(0, GCH)], osems.at[b]).wait()

    start_gather(0, 0)
    start_gather(1, 1)

    @pl.loop(0, KMAX - 2, step=NBUF)
    def _(k):
        for d in range(NBUF):
            kk = k + d
            b = d
            b3 = (d + 2) % NBUF
            wait_gather(kk, b)
            start_scatter(kk, b)

            @pl.when(jnp.logical_and(kk >= 1, kk + 2 < k_w))
            def _():
                drain_scatter(b3)

            @pl.when(kk + 2 < k_w)
            def _():
                start_gather(kk + 2, b3)

    wait_gather(KMAX - 2, 0)
    start_scatter(KMAX - 2, 0)

    @pl.when(k_w == KMAX)
    def _():
        wait_gather(KMAX - 1, 1)
        start_scatter(KMAX - 1, 1)

    for b in range(NBUF):
        drain_scatter(b)


def _gather(scaled, idx_pad):
    k = pl.kernel(
        _gather_body,
        out_type=jax.ShapeDtypeStruct((N_FINE, D_FEAT), jnp.float32),
        mesh=_mesh(),
        scratch_types=[
            pltpu.VMEM((KMAX * GCH,), jnp.int32),
            pltpu.VMEM((NBUF, GCH, D_FEAT), jnp.float32),
            pltpu.SemaphoreType.DMA((NBUF,)),
            pltpu.SemaphoreType.DMA((NBUF,)),
        ],
    )
    return k(scaled, idx_pad)


def kernel(features, cluster):
    idx = cluster.astype(jnp.int32)
    idx_pad = jnp.concatenate(
        [idx, jnp.full((HPAD - N_FINE,), N_COARSE, jnp.int32)])
    partials = _hist(idx_pad)
    pc0 = partials[0, :N_COARSE].reshape(N_COARSE, 1)
    pc1 = partials[1, :N_COARSE].reshape(N_COARSE, 1)
    scaled = _scale(features, pc0, pc1)
    return _gather(scaled, idx_pad)


# speedup vs baseline: 2.4442x; 1.2291x over previous
import jax
import jax.numpy as jnp
from jax import lax
from jax.experimental import pallas as pl
from jax.experimental.pallas import tpu as pltpu
from jax.experimental.pallas import tpu_sc as plsc

N_FINE = 50000
N_COARSE = 10000
D_FEAT = 512

NC, NS = 2, 16
NW = NC * NS

CW = 112
NCH_H = 14
HPASS = 2
CH_T = CW * NCH_H * HPASS
HPAD = NS * CH_T
NBINS = 10240
BSL = NBINS // NS

GCH = 80
NCHUNK = N_FINE // GCH
KMAX = -(-NCHUNK // NW)
NBUF = 3


def _body(feat_hbm, idx_hbm, out_hbm, inv_hbm,
          idx_all, hidx_v, ones_v, slc_v, scale_v, sbc_v, rows_v,
          hist_sh, psem, hsem, gsems, ssems, osems):
    cid = lax.axis_index("c")
    sid = lax.axis_index("s")
    wid = cid * NS + sid
    start_c = wid * 19 + jnp.minimum(wid, 17)
    k_w = jnp.where(wid < 17, KMAX, KMAX - 1)

    pltpu.async_copy(idx_hbm.at[pl.ds(start_c * GCH, KMAX * GCH)], idx_all,
                     psem)

    def fill(i, _):
        ones_v[0, pl.ds(i * 16, 16)] = jnp.ones((16,), jnp.float32)
        return 0

    lax.fori_loop(0, CW // 16, fill, 0)

    def fill_z(i, _):
        slc_v[pl.ds(i * 16, 16)] = jnp.zeros((16,), jnp.float32)
        return 0

    lax.fori_loop(0, BSL // 16, fill_z, 0)
    pltpu.sync_copy(slc_v, hist_sh.at[pl.ds(sid * BSL, BSL)])
    plsc.subcore_barrier()

    for p in range(HPASS):
        hbase = sid * CH_T + p * (NCH_H * CW)
        for j in range(NCH_H):
            pltpu.async_copy(idx_hbm.at[pl.ds(hbase + j * CW, CW)],
                             hidx_v.at[j], hsem)
        for j in range(NCH_H):
            pltpu.make_async_copy(idx_hbm.at[pl.ds(hbase + j * CW, CW)],
                                  hidx_v.at[j], hsem).wait()
        for j in range(NCH_H):
            pltpu.async_copy(ones_v.at[0], hist_sh.at[hidx_v.at[j]], hsem,
                             add=True)
        for j in range(NCH_H):
            pltpu.make_async_copy(ones_v.at[0], hist_sh.at[hidx_v.at[j]],
                                  hsem).wait()
    plsc.subcore_barrier()

    pltpu.sync_copy(hist_sh.at[pl.ds(sid * BSL, BSL)], slc_v)

    def inv_step(i, _):
        c = slc_v[pl.ds(i * 16, 16)]
        slc_v[pl.ds(i * 16, 16)] = 1.0 / jnp.maximum(c, 1.0)
        return 0

    lax.fori_loop(0, BSL // 16, inv_step, 0)
    pltpu.sync_copy(slc_v, inv_hbm.at[pl.ds(sid * BSL, BSL)])
    plsc.subcore_barrier()

    pltpu.make_async_copy(idx_hbm.at[pl.ds(start_c * GCH, KMAX * GCH)],
                          idx_all, psem).wait()

    def idx_slice(k):
        return idx_all.at[pl.ds(k * GCH, GCH)]

    def start_gather(k, b):
        pltpu.async_copy(feat_hbm.at[idx_slice(k)], rows_v.at[b], gsems.at[b])
        pltpu.async_copy(inv_hbm.at[idx_slice(k)], scale_v.at[b],
                         ssems.at[b])

    def consume(k, b):
        pltpu.make_async_copy(feat_hbm.at[idx_slice(k)], rows_v.at[b],
                              gsems.at[b]).wait()
        pltpu.make_async_copy(inv_hbm.at[idx_slice(k)],
                              scale_v.at[b], ssems.at[b]).wait()
        for g in range(GCH // 16):
            sv = scale_v[b, pl.ds(g * 16, 16)]
            for l in range(16):
                sbc_v[pl.ds((g * 16 + l) * 16, 16)] = jnp.full(
                    (16,), sv[l], jnp.float32)

        def row(r, _):
            sb = sbc_v[pl.ds(r * 16, 16)]
            for i in range(D_FEAT // 16):
                rows_v[b, r, pl.ds(i * 16, 16)] = (
                    rows_v[b, r, pl.ds(i * 16, 16)] * sb)
            return 0

        lax.fori_loop(0, GCH, row, 0)
        pltpu.async_copy(rows_v.at[b],
                         out_hbm.at[pl.ds((start_c + k) * GCH, GCH)],
                         osems.at[b])

    def drain_scatter(b):
        pltpu.make_async_copy(rows_v.at[b], out_hbm.at[pl.ds(0, GCH)],
                              osems.at[b]).wait()

    start_gather(0, 0)
    start_gather(1, 1)

    @pl.loop(0, KMAX - 2, step=NBUF)
    def _(k):
        for d in range(NBUF):
            kk = k + d
            b = d
            b3 = (d + 2) % NBUF
            consume(kk, b)

            @pl.when(jnp.logical_and(kk >= 1, kk + 2 < k_w))
            def _():
                drain_scatter(b3)

            @pl.when(kk + 2 < k_w)
            def _():
                start_gather(kk + 2, b3)

    consume(KMAX - 2, 0)

    @pl.when(k_w == KMAX)
    def _():
        consume(KMAX - 1, 1)

    for b in range(NBUF):
        drain_scatter(b)


def _fused(features, idx_pad):
    k = pl.kernel(
        _body,
        out_type=(jax.ShapeDtypeStruct((N_FINE, D_FEAT), jnp.float32),
                  jax.ShapeDtypeStruct((NBINS,), jnp.float32)),
        mesh=plsc.VectorSubcoreMesh(core_axis_name="c", subcore_axis_name="s",
                                    num_cores=NC, num_subcores=NS),
        scratch_types=[
            pltpu.VMEM((KMAX * GCH,), jnp.int32),
            pltpu.VMEM((NCH_H, CW), jnp.int32),
            pltpu.VMEM((1, CW), jnp.float32),
            pltpu.VMEM((BSL,), jnp.float32),
            pltpu.VMEM((NBUF, GCH), jnp.float32),
            pltpu.VMEM((GCH * 16,), jnp.float32),
            pltpu.VMEM((NBUF, GCH, D_FEAT), jnp.float32),
            pltpu.VMEM_SHARED((NBINS,), jnp.float32),
            pltpu.SemaphoreType.DMA,
            pltpu.SemaphoreType.DMA,
            pltpu.SemaphoreType.DMA((NBUF,)),
            pltpu.SemaphoreType.DMA((NBUF,)),
            pltpu.SemaphoreType.DMA((NBUF,)),
        ],
    )
    out, _ = k(features, idx_pad)
    return out


def kernel(features, cluster):
    idx = cluster.astype(jnp.int32)
    idx_pad = jnp.concatenate(
        [idx, jnp.full((HPAD - N_FINE,), N_COARSE, jnp.int32)])
    return _fused(features, idx_pad)
